# baseline (speedup 1.0000x reference)
import functools

import jax
import jax.numpy as jnp
from jax import lax
from jax.experimental import pallas as pl
from jax.experimental.pallas import tpu as pltpu
from jax.experimental.pallas import tpu_sc as plsc

K_SEG = 1024
L = 16


@functools.partial(jax.jit, static_argnums=(2, 3, 4))
def _sup_pix_pool(img, spx, B, C, HW):
    NC, NS = 2, 16
    NW = NC * NS
    G = NW // B
    CPG = C // G
    NV = HW // L
    ACC = L * K_SEG

    mesh = plsc.VectorSubcoreMesh(core_axis_name="c", subcore_axis_name="s")

    @functools.partial(
        pl.kernel,
        out_type=jax.ShapeDtypeStruct((B, C, K_SEG), jnp.float32),
        mesh=mesh,
        scratch_types=[
            pltpu.VMEM((HW,), jnp.int32),
            pltpu.VMEM((HW,), jnp.float32),
            pltpu.VMEM((ACC,), jnp.float32),
            pltpu.VMEM((K_SEG,), jnp.float32),
        ],
        compiler_params=pltpu.CompilerParams(needs_layout_passes=False),
    )
    def pool(img_hbm, spx_hbm, out_hbm, idx_ref, data_ref, acc_ref, row_ref):
        wid = lax.axis_index("s") * NC + lax.axis_index("c")
        b = wid // G
        g = wid % G

        pltpu.sync_copy(spx_hbm.at[b], idx_ref)
        lane_off = lax.iota(jnp.int32, L) * K_SEG

        def mk_idx(v, carry):
            sl = pl.ds(v * L, L)
            idx_ref[sl] = idx_ref[sl] + lane_off
            return carry

        lax.fori_loop(0, NV, mk_idx, 0)

        neg_inf = jnp.full((L,), -jnp.inf, dtype=jnp.float32)

        def per_channel(j, carry):
            ch = g * CPG + j
            pltpu.sync_copy(img_hbm.at[b, ch], data_ref)

            def init(v, c):
                acc_ref[pl.ds(v * L, L)] = neg_inf
                return c

            lax.fori_loop(0, ACC // L, init, 0)

            def scat(v, c):
                sl = pl.ds(v * L, L)
                ivec = idx_ref[sl]
                dvec = data_ref[sl]
                old = plsc.load_gather(acc_ref, [ivec])
                plsc.store_scatter(acc_ref, [ivec], jnp.maximum(old, dvec))
                return c

            lax.fori_loop(0, NV, scat, 0)

            def red(kv, c):
                m = acc_ref[pl.ds(kv * L, L)]
                for cpy in range(1, L):
                    m = jnp.maximum(m, acc_ref[pl.ds(cpy * K_SEG + kv * L, L)])
                row_ref[pl.ds(kv * L, L)] = m
                return c

            lax.fori_loop(0, K_SEG // L, red, 0)
            pltpu.sync_copy(row_ref, out_hbm.at[b, ch])
            return carry

        lax.fori_loop(0, CPG, per_channel, 0)

    return pool(img, spx)


def kernel(img, spx):
    B, C, H, W = img.shape
    HW = H * W
    img2 = img.reshape(B, C, HW)
    spx2 = spx.reshape(B, HW).astype(jnp.int32)
    return _sup_pix_pool(img2, spx2, B, C, HW)
